# baseline (speedup 1.0000x reference)
import jax
import jax.numpy as jnp
from jax.experimental import pallas as pl
from jax.experimental.pallas import tpu as pltpu

_BM = 200


def _gcn_body(adj_ref, x_ref, w1_ref, w23_ref,
              mu_ref, lv_ref, s_ref, h1_ref):
    p = pl.program_id(0)
    i = pl.program_id(1)

    @pl.when(jnp.logical_and(p == 0, i == 0))
    def _():
        s_ref[...] = jnp.dot(x_ref[...], w1_ref[...],
                             preferred_element_type=jnp.float32)

    @pl.when(jnp.logical_and(p == 1, i == 0))
    def _():
        s_ref[...] = jnp.dot(h1_ref[...], w23_ref[...],
                             preferred_element_type=jnp.float32)

    blk = jnp.maximum(
        jnp.dot(adj_ref[...], s_ref[...],
                preferred_element_type=jnp.float32), 0.0)

    @pl.when(p == 0)
    def _():
        h1_ref[pl.ds(i * _BM, _BM), :] = blk

    @pl.when(p == 1)
    def _():
        h = mu_ref.shape[1]
        mu_ref[...] = blk[:, :h]
        lv_ref[...] = blk[:, h:]


def kernel(x, adj, W1, W2, W3):
    n, d = x.shape
    h1w = W1.shape[1]
    h2 = W2.shape[1]
    nb = n // _BM
    w23 = jnp.concatenate([W2, W3], axis=1)

    out_idx = lambda p, i: (i * p + nb * (1 - p), 0)

    mu, logvar = pl.pallas_call(
        _gcn_body,
        grid=(2, nb),
        in_specs=[
            pl.BlockSpec((_BM, n), lambda p, i: (i, 0)),
            pl.BlockSpec((n, d), lambda p, i: (0, 0)),
            pl.BlockSpec((d, h1w), lambda p, i: (0, 0)),
            pl.BlockSpec((h1w, 2 * h2), lambda p, i: (0, 0)),
        ],
        out_specs=[
            pl.BlockSpec((_BM, h2), out_idx),
            pl.BlockSpec((_BM, h2), out_idx),
        ],
        out_shape=[
            jax.ShapeDtypeStruct((n + _BM, h2), jnp.float32),
            jax.ShapeDtypeStruct((n + _BM, h2), jnp.float32),
        ],
        scratch_shapes=[
            pltpu.VMEM((n, 2 * h2), jnp.float32),
            pltpu.VMEM((n, h1w), jnp.float32),
        ],
    )(adj, x, W1, w23)
    mu = mu[:n]
    logvar = logvar[:n]
    return (mu, mu, logvar)


# speedup vs baseline: 1.0359x; 1.0359x over previous
import jax
import jax.numpy as jnp
from jax.experimental import pallas as pl
from jax.experimental.pallas import tpu as pltpu

_BM = 400


def _gcn_body(adj_ref, x_ref, w1_ref, w23_ref,
              mu_ref, lv_ref, s_ref, h1_ref):
    p = pl.program_id(0)
    i = pl.program_id(1)

    @pl.when(jnp.logical_and(p == 0, i == 0))
    def _():
        s_ref[...] = jnp.dot(x_ref[...], w1_ref[...],
                             preferred_element_type=jnp.float32)

    @pl.when(jnp.logical_and(p == 1, i == 0))
    def _():
        s_ref[...] = jnp.dot(h1_ref[...], w23_ref[...],
                             preferred_element_type=jnp.float32)

    blk = jnp.maximum(
        jnp.dot(adj_ref[...], s_ref[...],
                preferred_element_type=jnp.float32), 0.0)

    @pl.when(p == 0)
    def _():
        h1_ref[pl.ds(i * _BM, _BM), :] = blk

    @pl.when(p == 1)
    def _():
        h = mu_ref.shape[1]
        mu_ref[...] = blk[:, :h]
        lv_ref[...] = blk[:, h:]


def kernel(x, adj, W1, W2, W3):
    n, d = x.shape
    h1w = W1.shape[1]
    h2 = W2.shape[1]
    nb = n // _BM
    w23 = jnp.concatenate([W2, W3], axis=1)

    out_idx = lambda p, i: (i * p, 0)

    mu, logvar = pl.pallas_call(
        _gcn_body,
        grid=(2, nb),
        in_specs=[
            pl.BlockSpec((_BM, n), lambda p, i: (i, 0)),
            pl.BlockSpec((n, d), lambda p, i: (0, 0)),
            pl.BlockSpec((d, h1w), lambda p, i: (0, 0)),
            pl.BlockSpec((h1w, 2 * h2), lambda p, i: (0, 0)),
        ],
        out_specs=[
            pl.BlockSpec((_BM, h2), out_idx),
            pl.BlockSpec((_BM, h2), out_idx),
        ],
        out_shape=[
            jax.ShapeDtypeStruct((n, h2), jnp.float32),
            jax.ShapeDtypeStruct((n, h2), jnp.float32),
        ],
        scratch_shapes=[
            pltpu.VMEM((n, 2 * h2), jnp.float32),
            pltpu.VMEM((n, h1w), jnp.float32),
        ],
    )(adj, x, W1, w23)
    return (mu, mu, logvar)
